# Initial kernel scaffold; baseline (speedup 1.0000x reference)
import jax
import jax.numpy as jnp
from jax.experimental import pallas as pl


def kernel(x, skip, W_trans, W_up, W1, W2, W3, edge_index):
    raise NotImplementedError("write your pallas kernel here")


# speedup vs baseline: 3.4352x; 3.4352x over previous
import functools

import jax
import jax.numpy as jnp
from jax import lax
from jax.experimental import pallas as pl
from jax.experimental.pallas import tpu as pltpu
from jax.experimental.pallas import tpu_sc as plsc

N = 10000
D = 128
E = 320000

NC = 2
NS = 16
NW = NC * NS
CHUNK = 128
K = 80
E_PAD = NW * K * CHUNK
ROWS_PER_TILE = 640
N_PAD = NS * ROWS_PER_TILE


def _sc_agg_body(y_hbm, src_hbm, dst_hbm, out_hbm, src_v, dst_v, rows_v, acc, sem):
  c = lax.axis_index("c")
  s = lax.axis_index("s")
  wid = s * NC + c
  base = s * ROWS_PER_TILE

  pltpu.sync_copy(src_hbm.at[wid], src_v)
  pltpu.sync_copy(dst_hbm.at[wid], dst_v)

  pltpu.sync_copy(y_hbm.at[pl.ds(base, ROWS_PER_TILE)],
                  acc.at[pl.ds(base, ROWS_PER_TILE)])
  plsc.subcore_barrier()

  @pl.loop(0, K)
  def _chunk(j):
    pltpu.async_copy(y_hbm.at[src_v.at[j]], rows_v, sem).wait()
    pltpu.sync_copy(rows_v, acc.at[dst_v.at[j]], add=True)

  plsc.subcore_barrier()
  pltpu.sync_copy(acc.at[pl.ds(base, ROWS_PER_TILE)],
                  out_hbm.at[c].at[pl.ds(base, ROWS_PER_TILE)])


_sc_agg = pl.kernel(
    _sc_agg_body,
    out_type=jax.ShapeDtypeStruct((NC, N_PAD, D), jnp.float32),
    mesh=plsc.VectorSubcoreMesh(core_axis_name="c", subcore_axis_name="s"),
    scratch_types=[
        pltpu.VMEM((K, CHUNK), jnp.int32),
        pltpu.VMEM((K, CHUNK), jnp.int32),
        pltpu.VMEM((CHUNK, D), jnp.float32),
        pltpu.VMEM_SHARED((N_PAD, D), jnp.float32),
        pltpu.SemaphoreType.DMA,
    ],
)


def _leaky_bn(h):
  mean = jnp.mean(h, axis=0, keepdims=True)
  var = jnp.mean((h - mean) * (h - mean), axis=0, keepdims=True)
  h = (h - mean) / jnp.sqrt(var + 1e-5)
  return jnp.where(h > 0, h, 0.01 * h)


def _tc_first_body(x_ref, w_ref, y_ref):
  y = jnp.dot(x_ref[...], w_ref[...], preferred_element_type=jnp.float32)
  y_ref[0:N, :] = y
  y_ref[N:N_PAD, :] = jnp.zeros((N_PAD - N, D), jnp.float32)


def _tc_first(x, w):
  return pl.pallas_call(
      _tc_first_body,
      out_shape=jax.ShapeDtypeStruct((N_PAD, D), jnp.float32),
  )(x, w)


def _tc_mid_body(p_ref, y_ref, skip_ref, w_ref, o_ref, *, bn):
  s = p_ref[0, 0:N, :] + p_ref[1, 0:N, :] - y_ref[0:N, :]
  if skip_ref is not None:
    s = s + skip_ref[...]
  if bn:
    s = _leaky_bn(s)
  o_ref[0:N, :] = jnp.dot(s, w_ref[...], preferred_element_type=jnp.float32)
  o_ref[N:N_PAD, :] = jnp.zeros((N_PAD - N, D), jnp.float32)


def _tc_mid(p, y, w, skip=None, bn=True):
  if skip is None:
    body = lambda p_ref, y_ref, w_ref, o_ref: _tc_mid_body(
        p_ref, y_ref, None, w_ref, o_ref, bn=bn)
    args = (p, y, w)
  else:
    body = functools.partial(_tc_mid_body, bn=bn)
    args = (p, y, skip, w)
  return pl.pallas_call(
      body,
      out_shape=jax.ShapeDtypeStruct((N_PAD, D), jnp.float32),
  )(*args)


def _tc_last_body(p_ref, y_ref, o_ref):
  o_ref[...] = _leaky_bn(p_ref[0, 0:N, :] + p_ref[1, 0:N, :] - y_ref[0:N, :])


def _tc_last(p, y):
  return pl.pallas_call(
      _tc_last_body,
      out_shape=jax.ShapeDtypeStruct((N, D), jnp.float32),
  )(p, y)


def kernel(x, skip, W_trans, W_up, W1, W2, W3, edge_index):
  src = edge_index[0].astype(jnp.int32)
  dst = edge_index[1].astype(jnp.int32)
  pad = E_PAD - E
  src_p = jnp.concatenate([src, jnp.full((pad,), N, jnp.int32)]).reshape(
      NW, K, CHUNK)
  dst_p = jnp.concatenate([dst, jnp.zeros((pad,), jnp.int32)]).reshape(
      NW, K, CHUNK)

  y1 = _tc_first(x, W_trans)
  p1 = _sc_agg(y1, src_p, dst_p)
  y2 = _tc_mid(p1, y1, W_up, bn=True)
  p2 = _sc_agg(y2, src_p, dst_p)
  y3 = _tc_mid(p2, y2, W1, skip=skip, bn=False)
  p3 = _sc_agg(y3, src_p, dst_p)
  y4 = _tc_mid(p3, y3, W2, bn=True)
  p4 = _sc_agg(y4, src_p, dst_p)
  y5 = _tc_mid(p4, y4, W3, bn=True)
  p5 = _sc_agg(y5, src_p, dst_p)
  return _tc_last(p5, y5)


# speedup vs baseline: 12.6345x; 3.6780x over previous
import functools

import jax
import jax.numpy as jnp
from jax import lax
from jax.experimental import pallas as pl
from jax.experimental.pallas import tpu as pltpu
from jax.experimental.pallas import tpu_sc as plsc

N = 10000
D = 128
E = 320000

NC = 2
NS = 16
NW = NC * NS
CHUNK = 128
K = 80
KH = 40
E_PAD = NW * K * CHUNK
ROWS_PER_TILE = 640
N_PAD = NS * ROWS_PER_TILE


def _sc_agg_body(y_hbm, src_hbm, dst_hbm, out_hbm, src_v, dst_v, rows0, rows1,
                 acc, sem0, sem1):
  c = lax.axis_index("c")
  s = lax.axis_index("s")
  wid = s * NC + c
  base = s * ROWS_PER_TILE

  pltpu.sync_copy(y_hbm.at[pl.ds(base, ROWS_PER_TILE)],
                  acc.at[pl.ds(base, ROWS_PER_TILE)])
  plsc.subcore_barrier()

  for ph in range(K // KH):
    off = ph * KH
    pltpu.sync_copy(src_hbm.at[wid].at[pl.ds(off, KH)], src_v)
    pltpu.sync_copy(dst_hbm.at[wid].at[pl.ds(off, KH)], dst_v)

    pltpu.async_copy(y_hbm.at[src_v.at[0]], rows0, sem0)
    pltpu.async_copy(y_hbm.at[src_v.at[1]], rows1, sem1)

    @pl.loop(0, KH, step=2)
    def _pair(j):
      pltpu.make_async_copy(y_hbm.at[src_v.at[j]], rows0, sem0).wait()
      pltpu.sync_copy(rows0, acc.at[dst_v.at[j]], add=True)

      @pl.when(j + 2 < KH)
      def _():
        pltpu.async_copy(y_hbm.at[src_v.at[j + 2]], rows0, sem0)

      pltpu.make_async_copy(y_hbm.at[src_v.at[j + 1]], rows1, sem1).wait()
      pltpu.sync_copy(rows1, acc.at[dst_v.at[j + 1]], add=True)

      @pl.when(j + 3 < KH)
      def _():
        pltpu.async_copy(y_hbm.at[src_v.at[j + 3]], rows1, sem1)

  plsc.subcore_barrier()
  pltpu.sync_copy(acc.at[pl.ds(base, ROWS_PER_TILE)],
                  out_hbm.at[c].at[pl.ds(base, ROWS_PER_TILE)])


_sc_agg = pl.kernel(
    _sc_agg_body,
    out_type=jax.ShapeDtypeStruct((NC, N_PAD, D), jnp.float32),
    mesh=plsc.VectorSubcoreMesh(core_axis_name="c", subcore_axis_name="s"),
    scratch_types=[
        pltpu.VMEM((KH, CHUNK), jnp.int32),
        pltpu.VMEM((KH, CHUNK), jnp.int32),
        pltpu.VMEM((CHUNK, D), jnp.float32),
        pltpu.VMEM((CHUNK, D), jnp.float32),
        pltpu.VMEM_SHARED((N_PAD, D), jnp.float32),
        pltpu.SemaphoreType.DMA,
        pltpu.SemaphoreType.DMA,
    ],
)


def _leaky_bn(h):
  mean = jnp.mean(h, axis=0, keepdims=True)
  var = jnp.mean((h - mean) * (h - mean), axis=0, keepdims=True)
  h = (h - mean) / jnp.sqrt(var + 1e-5)
  return jnp.where(h > 0, h, 0.01 * h)


def _tc_first_body(x_ref, w_ref, y_ref):
  y = jnp.dot(x_ref[...], w_ref[...], preferred_element_type=jnp.float32)
  y_ref[0:N, :] = y
  y_ref[N:N_PAD, :] = jnp.zeros((N_PAD - N, D), jnp.float32)


def _tc_first(x, w):
  return pl.pallas_call(
      _tc_first_body,
      out_shape=jax.ShapeDtypeStruct((N_PAD, D), jnp.float32),
  )(x, w)


def _tc_mid_body(p_ref, y_ref, skip_ref, w_ref, o_ref, *, bn):
  s = p_ref[0, 0:N, :] + p_ref[1, 0:N, :] - y_ref[0:N, :]
  if skip_ref is not None:
    s = s + skip_ref[...]
  if bn:
    s = _leaky_bn(s)
  o_ref[0:N, :] = jnp.dot(s, w_ref[...], preferred_element_type=jnp.float32)
  o_ref[N:N_PAD, :] = jnp.zeros((N_PAD - N, D), jnp.float32)


def _tc_mid(p, y, w, skip=None, bn=True):
  if skip is None:
    body = lambda p_ref, y_ref, w_ref, o_ref: _tc_mid_body(
        p_ref, y_ref, None, w_ref, o_ref, bn=bn)
    args = (p, y, w)
  else:
    body = functools.partial(_tc_mid_body, bn=bn)
    args = (p, y, skip, w)
  return pl.pallas_call(
      body,
      out_shape=jax.ShapeDtypeStruct((N_PAD, D), jnp.float32),
  )(*args)


def _tc_last_body(p_ref, y_ref, o_ref):
  o_ref[...] = _leaky_bn(p_ref[0, 0:N, :] + p_ref[1, 0:N, :] - y_ref[0:N, :])


def _tc_last(p, y):
  return pl.pallas_call(
      _tc_last_body,
      out_shape=jax.ShapeDtypeStruct((N, D), jnp.float32),
  )(p, y)


def kernel(x, skip, W_trans, W_up, W1, W2, W3, edge_index):
  src = edge_index[0].astype(jnp.int32)
  dst = edge_index[1].astype(jnp.int32)
  pad = E_PAD - E
  it = jnp.arange(pad, dtype=jnp.int32)
  src_p = jnp.concatenate([src, N + it % (N_PAD - N)]).reshape(
      K, NW, CHUNK).transpose(1, 0, 2)
  dst_p = jnp.concatenate([dst, it % N]).reshape(
      K, NW, CHUNK).transpose(1, 0, 2)

  y1 = _tc_first(x, W_trans)
  p1 = _sc_agg(y1, src_p, dst_p)
  y2 = _tc_mid(p1, y1, W_up, bn=True)
  p2 = _sc_agg(y2, src_p, dst_p)
  y3 = _tc_mid(p2, y2, W1, skip=skip, bn=False)
  p3 = _sc_agg(y3, src_p, dst_p)
  y4 = _tc_mid(p3, y3, W2, bn=True)
  p4 = _sc_agg(y4, src_p, dst_p)
  y5 = _tc_mid(p4, y4, W3, bn=True)
  p5 = _sc_agg(y5, src_p, dst_p)
  return _tc_last(p5, y5)
